# baseline (speedup 1.0000x reference)
import functools

import jax
import jax.numpy as jnp
from jax import lax
from jax.experimental import pallas as pl
from jax.experimental.pallas import tpu as pltpu
from jax.experimental.pallas import tpu_sc as plsc

N = 10000
D = 128
E = 320000
NC = 2
NS = 16
EPW = E // NS
CH = 80
NCHUNK = EPW // CH
NP = 10240
RPW = NP // NS
ZR = 128
B = 4096

_MESH = plsc.VectorSubcoreMesh(
    core_axis_name="c", subcore_axis_name="s", num_cores=NC, num_subcores=NS)

RB1 = E + CH
CAPE = 2 * (E + CH)
EPAD = 2 * CH
MS = 80
MAXPAIRS = (E + CH) // CH // NS // 2 + 2


def _pos_call(t2d):
  R = E // 128

  def body(t_ref, pos_ref, cnt_ref, pad_ref):
    m0 = (t_ref[...] == 0).astype(jnp.float32)
    m1 = 1.0 - m0
    iy = lax.broadcasted_iota(jnp.int32, (128, 128), 0)
    ix = lax.broadcasted_iota(jnp.int32, (128, 128), 1)
    sl128 = (iy < ix).astype(jnp.float32)
    ry = lax.broadcasted_iota(jnp.int32, (R, R), 0)
    rx = lax.broadcasted_iota(jnp.int32, (R, R), 1)
    tril = (rx < ry).astype(jnp.float32)

    def prefix(m):
      inrow = jnp.dot(m, sl128, preferred_element_type=jnp.float32)
      rt = jnp.sum(m, axis=1, keepdims=True)
      roff = jnp.dot(tril, rt, preferred_element_type=jnp.float32)
      return roff + inrow, jnp.sum(m)

    p0, n0 = prefix(m0)
    p1, n1 = prefix(m1)
    pos = jnp.where(t_ref[...] == 0, p0, RB1 + p1)
    pos_ref[...] = pos.astype(jnp.int32)
    n0i = n0.astype(jnp.int32)
    n1i = n1.astype(jnp.int32)
    cnt_ref[...] = jnp.concatenate(
        [jnp.full((1, 16), n0i), jnp.full((1, 16), n1i)], axis=0)
    ar = lax.broadcasted_iota(jnp.int32, (1, CH), 1)
    pad_ref[...] = jnp.concatenate([n0i + ar, RB1 + n1i + ar], axis=0)
  return pl.pallas_call(
      body,
      out_shape=(
          jax.ShapeDtypeStruct((E // 128, 128), jnp.int32),
          jax.ShapeDtypeStruct((NC, 16), jnp.int32),
          jax.ShapeDtypeStruct((NC, CH), jnp.int32),
      ),
  )(t2d)


@functools.partial(
    pl.kernel,
    out_type=(
        jax.ShapeDtypeStruct((CAPE,), jnp.int32),
        jax.ShapeDtypeStruct((CAPE,), jnp.int32),
        jax.ShapeDtypeStruct((CAPE,), jnp.float32),
    ),
    mesh=_MESH,
    scratch_types=[
        pltpu.VMEM((MS,), jnp.int32),
        pltpu.VMEM((MS,), jnp.int32),
        pltpu.VMEM((MS,), jnp.float32),
        pltpu.VMEM((MS,), jnp.int32),
        pltpu.SemaphoreType.DMA,
    ],
)
def _sc_scatter_meta(src_hbm, dst_hbm, w_hbm, pos_hbm, srcP, dstP, wP,
                     sv, dv, wv, pv, sem):
  c = lax.axis_index("c")
  s = lax.axis_index("s")
  wid = s * NC + c
  e0 = wid * (E // 32)

  def chunk(k, carry):
    off = e0 + k * MS
    pltpu.sync_copy(src_hbm.at[pl.ds(off, MS)], sv)
    pltpu.sync_copy(dst_hbm.at[pl.ds(off, MS)], dv)
    pltpu.sync_copy(w_hbm.at[pl.ds(off, MS)], wv)
    pltpu.sync_copy(pos_hbm.at[pl.ds(off, MS)], pv)
    pltpu.async_copy(sv, srcP.at[pv], sem)
    pltpu.make_async_copy(sv, srcP.at[pv], sem).wait()
    pltpu.async_copy(dv, dstP.at[pv], sem)
    pltpu.make_async_copy(dv, dstP.at[pv], sem).wait()
    pltpu.async_copy(wv, wP.at[pv], sem)
    pltpu.make_async_copy(wv, wP.at[pv], sem).wait()
    return carry
  lax.fori_loop(0, E // 32 // MS, chunk, 0)
  @pl.when(wid == 31)
  def _():
    for t in range(EPAD // MS):
      off = E + t * MS
      pltpu.sync_copy(src_hbm.at[pl.ds(off, MS)], sv)
      pltpu.sync_copy(dst_hbm.at[pl.ds(off, MS)], dv)
      pltpu.sync_copy(w_hbm.at[pl.ds(off, MS)], wv)
      pltpu.sync_copy(pos_hbm.at[pl.ds(off, MS)], pv)
      pltpu.async_copy(sv, srcP.at[pv], sem)
      pltpu.make_async_copy(sv, srcP.at[pv], sem).wait()
      pltpu.async_copy(dv, dstP.at[pv], sem)
      pltpu.make_async_copy(dv, dstP.at[pv], sem).wait()
      pltpu.async_copy(wv, wP.at[pv], sem)
      pltpu.make_async_copy(wv, wP.at[pv], sem).wait()


@functools.partial(
    pl.kernel,
    out_type=jax.ShapeDtypeStruct((NC, NP, D), jnp.float32),
    mesh=_MESH,
    scratch_types=(
        [pltpu.VMEM((CH,), jnp.int32) for _ in range(2)]
        + [pltpu.VMEM((CH,), jnp.int32) for _ in range(2)]
        + [pltpu.VMEM((CH,), jnp.float32) for _ in range(2)]
        + [pltpu.VMEM((CH, D), jnp.float32) for _ in range(2)]
        + [
            pltpu.VMEM((ZR, D), jnp.float32),
            pltpu.VMEM((16,), jnp.int32),
            pltpu.VMEM_SHARED((NP, D), jnp.float32),
        ]
        + [pltpu.SemaphoreType.DMA for _ in range(6)]
    ),
)
def _sc_aggregate(x_hbm, src_hbm, dst_hbm, w_hbm, cnt_hbm, out_hbm,
                  sia, sib, dia, dib, wva, wvb, ra, rb,
                  stage_v, cv_v, acc_sh, ga, gb, ma, mb, sa, sb):
  c = lax.axis_index("c")
  s = lax.axis_index("s")

  def zrow(i, carry):
    for j in range(D // 16):
      stage_v[i, pl.ds(j * 16, 16)] = jnp.zeros((16,), jnp.float32)
    return carry
  lax.fori_loop(0, ZR, zrow, 0)
  r0 = s * RPW
  for z in range(RPW // ZR):
    pltpu.sync_copy(stage_v, acc_sh.at[pl.ds(r0 + z * ZR, ZR)])
  plsc.subcore_barrier()

  pltpu.sync_copy(cnt_hbm.at[c], cv_v)
  ne = cv_v[...][0]
  nck = (ne + CH - 1) // CH
  q = nck // NS
  r = nck % NS
  my_n = q + (s < r).astype(jnp.int32)
  my_start = s * q + jnp.minimum(s, r)

  sets = ((sia, dia, wva, ra, ga, ma, sa),
          (sib, dib, wvb, rb, gb, mb, sb))

  def issue_meta(k, st):
    si_r, di_r, wv_r, _, _, msem, _ = st
    off = c * RB1 + (my_start + k) * CH
    pltpu.async_copy(src_hbm.at[pl.ds(off, CH)], si_r, msem)
    pltpu.async_copy(dst_hbm.at[pl.ds(off, CH)], di_r, msem)
    pltpu.async_copy(w_hbm.at[pl.ds(off, CH)], wv_r, msem)

  def wait_meta(st):
    si_r, di_r, wv_r, _, _, msem, _ = st
    pltpu.make_async_copy(src_hbm.at[pl.ds(0, CH)], si_r, msem).wait()
    pltpu.make_async_copy(dst_hbm.at[pl.ds(0, CH)], di_r, msem).wait()
    pltpu.make_async_copy(w_hbm.at[pl.ds(0, CH)], wv_r, msem).wait()

  def issue_gather(st):
    si_r, _, _, rows_r, gsem, _, _ = st
    pltpu.async_copy(x_hbm.at[si_r], rows_r, gsem)

  def wait_gather(st):
    si_r, _, _, rows_r, gsem, _, _ = st
    pltpu.make_async_copy(x_hbm.at[si_r], rows_r, gsem).wait()

  def issue_scatter(st):
    _, di_r, _, rows_r, _, _, ssem = st
    pltpu.async_copy(rows_r, acc_sh.at[di_r], ssem, add=True)

  def wait_scatter(st):
    _, di_r, _, rows_r, _, _, ssem = st
    pltpu.make_async_copy(rows_r, acc_sh.at[di_r], ssem).wait()

  def scale(st):
    _, _, wv_r, rows_r, _, _, _ = st
    for g in range(CH // 16):
      wg = wv_r[pl.ds(g * 16, 16)]
      for i in range(16):
        ws = wg[i]
        e = g * 16 + i
        for j in range(D // 16):
          sl = pl.ds(j * 16, 16)
          rows_r[e, sl] = rows_r[e, sl] * ws

  @pl.when(my_n >= 1)
  def _():
    issue_meta(0, sets[0])
    wait_meta(sets[0])
    issue_gather(sets[0])
  @pl.when(my_n >= 2)
  def _():
    issue_meta(1, sets[1])

  def pair(p, carry):
    for h in range(2):
      k = 2 * p + h
      sA = sets[h]
      sB = sets[1 - h]
      @pl.when(k < my_n)
      def _():
        wait_gather(sA)
        scale(sA)
        issue_scatter(sA)
        @pl.when(k + 2 < my_n)
        def _():
          wait_scatter(sA)
          issue_meta(k + 2, sA)
        @pl.when(k + 1 < my_n)
        def _():
          wait_meta(sB)
          issue_gather(sB)
    return carry
  lax.fori_loop(0, MAXPAIRS, pair, 0)
  @pl.when(my_n >= 2)
  def _():
    wait_scatter(sets[0])
    wait_scatter(sets[1])
  @pl.when(my_n == 1)
  def _():
    wait_scatter(sets[0])

  plsc.subcore_barrier()
  for z in range(RPW // ZR):
    rr = r0 + z * ZR
    pltpu.sync_copy(acc_sh.at[pl.ds(rr, ZR)], out_hbm.at[c, pl.ds(rr, ZR)])


_GB = B * 2 // (NC * NS)


@functools.partial(
    pl.kernel,
    out_type=jax.ShapeDtypeStruct((2 * B, D), jnp.float32),
    mesh=_MESH,
    scratch_types=[
        pltpu.VMEM((128,), jnp.int32),
        pltpu.VMEM((128, D), jnp.float32),
        pltpu.SemaphoreType.DMA,
    ],
)
def _sc_gather_rows(x_hbm, idx_hbm, out_hbm, idx_v, rows_v, sem):
  c = lax.axis_index("c")
  s = lax.axis_index("s")
  base = (s * NC + c) * _GB
  for t in range(_GB // 128):
    off = base + t * 128
    pltpu.sync_copy(idx_hbm.at[pl.ds(off, 128)], idx_v)
    pltpu.async_copy(x_hbm.at[idx_v], rows_v, sem).wait()
    pltpu.sync_copy(rows_v, out_hbm.at[pl.ds(off, 128)])


def _wsel_call(w2d, t2d):
  def body(w_ref, t_ref, o_ref):
    w = w_ref[...]
    t = t_ref[...]
    o_ref[0] = jnp.where(t == 0, w, 0.0)
    o_ref[1] = jnp.where(t == 1, w, 0.0)
  return pl.pallas_call(
      body,
      out_shape=jax.ShapeDtypeStruct((2, E // 128, 128), jnp.float32),
  )(w2d, t2d)


RB = 1000


def _tc_layer_call(A, x, relw, rootw, bias, g, b2, do_ln):
  def body(a_ref, x_ref, rw_ref, rootw_ref, bias_ref, g_ref, b2_ref, o_ref):
    y = jnp.dot(a_ref[0], rw_ref[0], preferred_element_type=jnp.float32)
    y = y + jnp.dot(a_ref[1], rw_ref[1], preferred_element_type=jnp.float32)
    y = y + jnp.dot(x_ref[...], rootw_ref[...],
                    preferred_element_type=jnp.float32)
    y = y + bias_ref[...]
    if do_ln:
      y = jnp.maximum(y, 0.0)
      m = jnp.mean(y, axis=-1, keepdims=True)
      yc = y - m
      v = jnp.mean(yc * yc, axis=-1, keepdims=True)
      y = yc * lax.rsqrt(v + 1e-5) * g_ref[...] + b2_ref[...]
    o_ref[...] = y
  return pl.pallas_call(
      body,
      grid=(N // RB,),
      in_specs=[
          pl.BlockSpec((2, RB, D), lambda i: (0, i, 0)),
          pl.BlockSpec((RB, D), lambda i: (i, 0)),
          pl.BlockSpec((2, D, D), lambda i: (0, 0, 0)),
          pl.BlockSpec((D, D), lambda i: (0, 0)),
          pl.BlockSpec((1, D), lambda i: (0, 0)),
          pl.BlockSpec((1, D), lambda i: (0, 0)),
          pl.BlockSpec((1, D), lambda i: (0, 0)),
      ],
      out_specs=pl.BlockSpec((RB, D), lambda i: (i, 0)),
      out_shape=jax.ShapeDtypeStruct((N, D), jnp.float32),
  )(A, x, relw, rootw, bias, g, b2)


def _tc_head_call(ui, w0, b0, w1, b1, w2, b2, owt, ob):
  def body(ui_ref, w0_ref, b0_ref, w1_ref, b1_ref, w2_ref, b2_ref,
           ow_ref, ob_ref, o_ref):
    u = ui_ref[:B]
    it = ui_ref[B:]
    h = (jnp.dot(u, w0_ref[:D], preferred_element_type=jnp.float32)
         + jnp.dot(it, w0_ref[D:], preferred_element_type=jnp.float32)
         + b0_ref[...])
    h = jnp.maximum(h, 0.0)
    h = jnp.maximum(
        jnp.dot(h, w1_ref[...], preferred_element_type=jnp.float32)
        + b1_ref[...], 0.0)
    h = jnp.maximum(
        jnp.dot(h, w2_ref[...], preferred_element_type=jnp.float32)
        + b2_ref[...], 0.0)
    nu = jnp.maximum(jnp.sqrt(jnp.sum(u * u, axis=-1, keepdims=True)), 1e-12)
    ni = jnp.maximum(jnp.sqrt(jnp.sum(it * it, axis=-1, keepdims=True)),
                     1e-12)
    gmf = (u / nu) * (it / ni)
    logit = (jnp.sum(gmf * ow_ref[:, :D], axis=-1, keepdims=True)
             + jnp.sum(h * ow_ref[:, D:], axis=-1, keepdims=True)
             + ob_ref[...])
    o_ref[...] = jax.nn.sigmoid(logit)
  return pl.pallas_call(
      body,
      out_shape=jax.ShapeDtypeStruct((B, 1), jnp.float32),
  )(ui, w0, b0, w1, b1, w2, b2, owt, ob)


def kernel(user_indices, item_indices, edge_index, edge_type, edge_weight,
           emb_table, rel_w0, rel_w1, rel_w2, root_w0, root_w1, root_w2,
           bias0, bias1, bias2, ln1_g, ln1_b, ln2_g, ln2_b,
           mlp_w0, mlp_b0, mlp_w1, mlp_b1, mlp_w2, mlp_b2, out_w, out_b):
  zi = jnp.zeros((EPAD,), jnp.int32)
  src1 = jnp.concatenate([edge_index[0].astype(jnp.int32), zi])
  dst1 = jnp.concatenate([edge_index[1].astype(jnp.int32), zi])
  w1 = jnp.concatenate([edge_weight, jnp.zeros((EPAD,), jnp.float32)])
  t2d = edge_type.astype(jnp.int32).reshape(E // 128, 128)
  pos2d, cntC, padpos = _pos_call(t2d)
  pos1 = jnp.concatenate([pos2d.reshape(E), padpos.reshape(EPAD)])
  srcC, dstC, wC = _sc_scatter_meta(src1, dst1, w1, pos1)

  x = emb_table
  layers = [
      (rel_w0, root_w0, bias0, ln1_g, ln1_b, True),
      (rel_w1, root_w1, bias1, ln2_g, ln2_b, True),
      (rel_w2, root_w2, bias2, ln2_g, ln2_b, False),
  ]
  for relw, rootw, bias, g, b2, do_ln in layers:
    A = _sc_aggregate(x, srcC, dstC, wC, cntC)
    x = _tc_layer_call(A, x, relw, rootw.reshape(D, D),
                       bias.reshape(1, D), g.reshape(1, D),
                       b2.reshape(1, D), do_ln)

  idx = jnp.concatenate([user_indices, item_indices]).astype(jnp.int32)
  ui = _sc_gather_rows(x, idx)
  out = _tc_head_call(
      ui, mlp_w0, mlp_b0.reshape(1, -1), mlp_w1, mlp_b1.reshape(1, -1),
      mlp_w2, mlp_b2.reshape(1, -1), out_w.reshape(1, -1),
      out_b.reshape(1, 1))
  return out.reshape(B)


# speedup vs baseline: 1.0073x; 1.0073x over previous
import functools

import jax
import jax.numpy as jnp
from jax import lax
from jax.experimental import pallas as pl
from jax.experimental.pallas import tpu as pltpu
from jax.experimental.pallas import tpu_sc as plsc

N = 10000
D = 128
E = 320000
NC = 2
NS = 16
EPW = E // NS
CH = 80
NCHUNK = EPW // CH
NP = 10240
RPW = NP // NS
ZR = 128
B = 4096

_MESH = plsc.VectorSubcoreMesh(
    core_axis_name="c", subcore_axis_name="s", num_cores=NC, num_subcores=NS)

RB1 = E + CH
CAPE = 2 * (E + CH)
EPAD = 2 * CH
MS = 80
MAXPAIRS = (E + CH) // CH // NS // 2 + 2


def _pos_call(t2d):
  R = E // 128

  def body(t_ref, pos_ref, cnt_ref, pad_ref):
    m0 = (t_ref[...] == 0).astype(jnp.float32)
    m1 = 1.0 - m0
    iy = lax.broadcasted_iota(jnp.int32, (128, 128), 0)
    ix = lax.broadcasted_iota(jnp.int32, (128, 128), 1)
    sl128 = (iy < ix).astype(jnp.float32)
    ry = lax.broadcasted_iota(jnp.int32, (R, R), 0)
    rx = lax.broadcasted_iota(jnp.int32, (R, R), 1)
    tril = (rx < ry).astype(jnp.float32)

    def prefix(m):
      inrow = jnp.dot(m, sl128, preferred_element_type=jnp.float32)
      rt = jnp.sum(m, axis=1, keepdims=True)
      roff = jnp.dot(tril, rt, preferred_element_type=jnp.float32)
      return roff + inrow, jnp.sum(m)

    p0, n0 = prefix(m0)
    p1, n1 = prefix(m1)
    pos = jnp.where(t_ref[...] == 0, p0, RB1 + p1)
    pos_ref[...] = pos.astype(jnp.int32)
    n0i = n0.astype(jnp.int32)
    n1i = n1.astype(jnp.int32)
    cnt_ref[...] = jnp.concatenate(
        [jnp.full((1, 16), n0i), jnp.full((1, 16), n1i)], axis=0)
    ar = lax.broadcasted_iota(jnp.int32, (1, CH), 1)
    pad_ref[...] = jnp.concatenate([n0i + ar, RB1 + n1i + ar], axis=0)
  return pl.pallas_call(
      body,
      out_shape=(
          jax.ShapeDtypeStruct((E // 128, 128), jnp.int32),
          jax.ShapeDtypeStruct((NC, 16), jnp.int32),
          jax.ShapeDtypeStruct((NC, CH), jnp.int32),
      ),
  )(t2d)


@functools.partial(
    pl.kernel,
    out_type=(
        jax.ShapeDtypeStruct((CAPE,), jnp.int32),
        jax.ShapeDtypeStruct((CAPE,), jnp.int32),
        jax.ShapeDtypeStruct((CAPE,), jnp.float32),
    ),
    mesh=_MESH,
    scratch_types=(
        [pltpu.VMEM((MS,), jnp.int32) for _ in range(2)]
        + [pltpu.VMEM((MS,), jnp.int32) for _ in range(2)]
        + [pltpu.VMEM((MS,), jnp.float32) for _ in range(2)]
        + [pltpu.VMEM((MS,), jnp.int32) for _ in range(2)]
        + [pltpu.SemaphoreType.DMA for _ in range(4)]
    ),
)
def _sc_scatter_meta(src_hbm, dst_hbm, w_hbm, pos_hbm, srcP, dstP, wP,
                     sv0, sv1, dv0, dv1, wv0, wv1, pv0, pv1,
                     is0, is1, os0, os1):
  c = lax.axis_index("c")
  s = lax.axis_index("s")
  wid = s * NC + c
  e0 = wid * (E // 32)
  NCK3 = E // 32 // MS
  sets = ((sv0, dv0, wv0, pv0, is0, os0), (sv1, dv1, wv1, pv1, is1, os1))

  def issue_in(k, st):
    sv, dv, wv, pv, isem, _ = st
    off = e0 + k * MS
    pltpu.async_copy(src_hbm.at[pl.ds(off, MS)], sv, isem)
    pltpu.async_copy(dst_hbm.at[pl.ds(off, MS)], dv, isem)
    pltpu.async_copy(w_hbm.at[pl.ds(off, MS)], wv, isem)
    pltpu.async_copy(pos_hbm.at[pl.ds(off, MS)], pv, isem)

  def wait_in(st):
    sv, dv, wv, pv, isem, _ = st
    pltpu.make_async_copy(src_hbm.at[pl.ds(0, MS)], sv, isem).wait()
    pltpu.make_async_copy(dst_hbm.at[pl.ds(0, MS)], dv, isem).wait()
    pltpu.make_async_copy(w_hbm.at[pl.ds(0, MS)], wv, isem).wait()
    pltpu.make_async_copy(pos_hbm.at[pl.ds(0, MS)], pv, isem).wait()

  def issue_out(st):
    sv, dv, wv, pv, _, osem = st
    pltpu.async_copy(sv, srcP.at[pv], osem)
    pltpu.async_copy(dv, dstP.at[pv], osem)
    pltpu.async_copy(wv, wP.at[pv], osem)

  def wait_out(st):
    sv, dv, wv, pv, _, osem = st
    pltpu.make_async_copy(sv, srcP.at[pv], osem).wait()
    pltpu.make_async_copy(dv, dstP.at[pv], osem).wait()
    pltpu.make_async_copy(wv, wP.at[pv], osem).wait()

  issue_in(0, sets[0])
  def pair(p, carry):
    for h in range(2):
      k = 2 * p + h
      sA = sets[h]
      sB = sets[1 - h]
      wait_in(sA)
      @pl.when(k >= 1)
      def _():
        wait_out(sB)
      @pl.when(k + 1 < NCK3)
      def _():
        issue_in(k + 1, sB)
      issue_out(sA)
    return carry
  lax.fori_loop(0, NCK3 // 2, pair, 0)
  wait_in(sets[0])
  wait_out(sets[1])
  issue_out(sets[0])
  wait_out(sets[0])
  @pl.when(wid == 31)
  def _():
    for t in range(EPAD // MS):
      off = E + t * MS
      st = sets[0]
      sv, dv, wv, pv, isem, _ = st
      pltpu.async_copy(src_hbm.at[pl.ds(off, MS)], sv, isem)
      pltpu.async_copy(dst_hbm.at[pl.ds(off, MS)], dv, isem)
      pltpu.async_copy(w_hbm.at[pl.ds(off, MS)], wv, isem)
      pltpu.async_copy(pos_hbm.at[pl.ds(off, MS)], pv, isem)
      wait_in(st)
      issue_out(st)
      wait_out(st)


@functools.partial(
    pl.kernel,
    out_type=jax.ShapeDtypeStruct((NC, NP, D), jnp.float32),
    mesh=_MESH,
    scratch_types=(
        [pltpu.VMEM((CH,), jnp.int32) for _ in range(2)]
        + [pltpu.VMEM((CH,), jnp.int32) for _ in range(2)]
        + [pltpu.VMEM((CH,), jnp.float32) for _ in range(2)]
        + [pltpu.VMEM((CH, D), jnp.float32) for _ in range(2)]
        + [
            pltpu.VMEM((ZR, D), jnp.float32),
            pltpu.VMEM((16,), jnp.int32),
            pltpu.VMEM_SHARED((NP, D), jnp.float32),
        ]
        + [pltpu.SemaphoreType.DMA for _ in range(6)]
    ),
)
def _sc_aggregate(x_hbm, src_hbm, dst_hbm, w_hbm, cnt_hbm, out_hbm,
                  sia, sib, dia, dib, wva, wvb, ra, rb,
                  stage_v, cv_v, acc_sh, ga, gb, ma, mb, sa, sb):
  c = lax.axis_index("c")
  s = lax.axis_index("s")

  def zrow(i, carry):
    for j in range(D // 16):
      stage_v[i, pl.ds(j * 16, 16)] = jnp.zeros((16,), jnp.float32)
    return carry
  lax.fori_loop(0, ZR, zrow, 0)
  r0 = s * RPW
  for z in range(RPW // ZR):
    pltpu.sync_copy(stage_v, acc_sh.at[pl.ds(r0 + z * ZR, ZR)])
  plsc.subcore_barrier()

  pltpu.sync_copy(cnt_hbm.at[c], cv_v)
  ne = cv_v[...][0]
  nck = (ne + CH - 1) // CH
  q = nck // NS
  r = nck % NS
  my_n = q + (s < r).astype(jnp.int32)
  my_start = s * q + jnp.minimum(s, r)

  sets = ((sia, dia, wva, ra, ga, ma, sa),
          (sib, dib, wvb, rb, gb, mb, sb))

  def issue_meta(k, st):
    si_r, di_r, wv_r, _, _, msem, _ = st
    off = c * RB1 + (my_start + k) * CH
    pltpu.async_copy(src_hbm.at[pl.ds(off, CH)], si_r, msem)
    pltpu.async_copy(dst_hbm.at[pl.ds(off, CH)], di_r, msem)
    pltpu.async_copy(w_hbm.at[pl.ds(off, CH)], wv_r, msem)

  def wait_meta(st):
    si_r, di_r, wv_r, _, _, msem, _ = st
    pltpu.make_async_copy(src_hbm.at[pl.ds(0, CH)], si_r, msem).wait()
    pltpu.make_async_copy(dst_hbm.at[pl.ds(0, CH)], di_r, msem).wait()
    pltpu.make_async_copy(w_hbm.at[pl.ds(0, CH)], wv_r, msem).wait()

  def issue_gather(st):
    si_r, _, _, rows_r, gsem, _, _ = st
    pltpu.async_copy(x_hbm.at[si_r], rows_r, gsem)

  def wait_gather(st):
    si_r, _, _, rows_r, gsem, _, _ = st
    pltpu.make_async_copy(x_hbm.at[si_r], rows_r, gsem).wait()

  def issue_scatter(st):
    _, di_r, _, rows_r, _, _, ssem = st
    pltpu.async_copy(rows_r, acc_sh.at[di_r], ssem, add=True)

  def wait_scatter(st):
    _, di_r, _, rows_r, _, _, ssem = st
    pltpu.make_async_copy(rows_r, acc_sh.at[di_r], ssem).wait()

  def scale(st):
    _, _, wv_r, rows_r, _, _, _ = st
    for g in range(CH // 16):
      wg = wv_r[pl.ds(g * 16, 16)]
      for i in range(16):
        ws = wg[i]
        e = g * 16 + i
        for j in range(D // 16):
          sl = pl.ds(j * 16, 16)
          rows_r[e, sl] = rows_r[e, sl] * ws

  @pl.when(my_n >= 1)
  def _():
    issue_meta(0, sets[0])
    wait_meta(sets[0])
    issue_gather(sets[0])
  @pl.when(my_n >= 2)
  def _():
    issue_meta(1, sets[1])

  def pair(p, carry):
    for h in range(2):
      k = 2 * p + h
      sA = sets[h]
      sB = sets[1 - h]
      @pl.when(k < my_n)
      def _():
        wait_gather(sA)
        scale(sA)
        issue_scatter(sA)
        @pl.when(k + 2 < my_n)
        def _():
          wait_scatter(sA)
          issue_meta(k + 2, sA)
        @pl.when(k + 1 < my_n)
        def _():
          wait_meta(sB)
          issue_gather(sB)
    return carry
  lax.fori_loop(0, MAXPAIRS, pair, 0)
  @pl.when(my_n >= 2)
  def _():
    wait_scatter(sets[0])
    wait_scatter(sets[1])
  @pl.when(my_n == 1)
  def _():
    wait_scatter(sets[0])

  plsc.subcore_barrier()
  for z in range(RPW // ZR):
    rr = r0 + z * ZR
    pltpu.sync_copy(acc_sh.at[pl.ds(rr, ZR)], out_hbm.at[c, pl.ds(rr, ZR)])


_GB = B * 2 // (NC * NS)


@functools.partial(
    pl.kernel,
    out_type=jax.ShapeDtypeStruct((2 * B, D), jnp.float32),
    mesh=_MESH,
    scratch_types=[
        pltpu.VMEM((128,), jnp.int32),
        pltpu.VMEM((128, D), jnp.float32),
        pltpu.SemaphoreType.DMA,
    ],
)
def _sc_gather_rows(x_hbm, idx_hbm, out_hbm, idx_v, rows_v, sem):
  c = lax.axis_index("c")
  s = lax.axis_index("s")
  base = (s * NC + c) * _GB
  for t in range(_GB // 128):
    off = base + t * 128
    pltpu.sync_copy(idx_hbm.at[pl.ds(off, 128)], idx_v)
    pltpu.async_copy(x_hbm.at[idx_v], rows_v, sem).wait()
    pltpu.sync_copy(rows_v, out_hbm.at[pl.ds(off, 128)])


def _wsel_call(w2d, t2d):
  def body(w_ref, t_ref, o_ref):
    w = w_ref[...]
    t = t_ref[...]
    o_ref[0] = jnp.where(t == 0, w, 0.0)
    o_ref[1] = jnp.where(t == 1, w, 0.0)
  return pl.pallas_call(
      body,
      out_shape=jax.ShapeDtypeStruct((2, E // 128, 128), jnp.float32),
  )(w2d, t2d)


RB = 1000


def _tc_layer_call(A, x, relw, rootw, bias, g, b2, do_ln):
  def body(a_ref, x_ref, rw_ref, rootw_ref, bias_ref, g_ref, b2_ref, o_ref):
    y = jnp.dot(a_ref[0], rw_ref[0], preferred_element_type=jnp.float32)
    y = y + jnp.dot(a_ref[1], rw_ref[1], preferred_element_type=jnp.float32)
    y = y + jnp.dot(x_ref[...], rootw_ref[...],
                    preferred_element_type=jnp.float32)
    y = y + bias_ref[...]
    if do_ln:
      y = jnp.maximum(y, 0.0)
      m = jnp.mean(y, axis=-1, keepdims=True)
      yc = y - m
      v = jnp.mean(yc * yc, axis=-1, keepdims=True)
      y = yc * lax.rsqrt(v + 1e-5) * g_ref[...] + b2_ref[...]
    o_ref[...] = y
  return pl.pallas_call(
      body,
      grid=(N // RB,),
      in_specs=[
          pl.BlockSpec((2, RB, D), lambda i: (0, i, 0)),
          pl.BlockSpec((RB, D), lambda i: (i, 0)),
          pl.BlockSpec((2, D, D), lambda i: (0, 0, 0)),
          pl.BlockSpec((D, D), lambda i: (0, 0)),
          pl.BlockSpec((1, D), lambda i: (0, 0)),
          pl.BlockSpec((1, D), lambda i: (0, 0)),
          pl.BlockSpec((1, D), lambda i: (0, 0)),
      ],
      out_specs=pl.BlockSpec((RB, D), lambda i: (i, 0)),
      out_shape=jax.ShapeDtypeStruct((N, D), jnp.float32),
  )(A, x, relw, rootw, bias, g, b2)


def _tc_head_call(ui, w0, b0, w1, b1, w2, b2, owt, ob):
  def body(ui_ref, w0_ref, b0_ref, w1_ref, b1_ref, w2_ref, b2_ref,
           ow_ref, ob_ref, o_ref):
    u = ui_ref[:B]
    it = ui_ref[B:]
    h = (jnp.dot(u, w0_ref[:D], preferred_element_type=jnp.float32)
         + jnp.dot(it, w0_ref[D:], preferred_element_type=jnp.float32)
         + b0_ref[...])
    h = jnp.maximum(h, 0.0)
    h = jnp.maximum(
        jnp.dot(h, w1_ref[...], preferred_element_type=jnp.float32)
        + b1_ref[...], 0.0)
    h = jnp.maximum(
        jnp.dot(h, w2_ref[...], preferred_element_type=jnp.float32)
        + b2_ref[...], 0.0)
    nu = jnp.maximum(jnp.sqrt(jnp.sum(u * u, axis=-1, keepdims=True)), 1e-12)
    ni = jnp.maximum(jnp.sqrt(jnp.sum(it * it, axis=-1, keepdims=True)),
                     1e-12)
    gmf = (u / nu) * (it / ni)
    logit = (jnp.sum(gmf * ow_ref[:, :D], axis=-1, keepdims=True)
             + jnp.sum(h * ow_ref[:, D:], axis=-1, keepdims=True)
             + ob_ref[...])
    o_ref[...] = jax.nn.sigmoid(logit)
  return pl.pallas_call(
      body,
      out_shape=jax.ShapeDtypeStruct((B, 1), jnp.float32),
  )(ui, w0, b0, w1, b1, w2, b2, owt, ob)


def kernel(user_indices, item_indices, edge_index, edge_type, edge_weight,
           emb_table, rel_w0, rel_w1, rel_w2, root_w0, root_w1, root_w2,
           bias0, bias1, bias2, ln1_g, ln1_b, ln2_g, ln2_b,
           mlp_w0, mlp_b0, mlp_w1, mlp_b1, mlp_w2, mlp_b2, out_w, out_b):
  zi = jnp.zeros((EPAD,), jnp.int32)
  src1 = jnp.concatenate([edge_index[0].astype(jnp.int32), zi])
  dst1 = jnp.concatenate([edge_index[1].astype(jnp.int32), zi])
  w1 = jnp.concatenate([edge_weight, jnp.zeros((EPAD,), jnp.float32)])
  t2d = edge_type.astype(jnp.int32).reshape(E // 128, 128)
  pos2d, cntC, padpos = _pos_call(t2d)
  pos1 = jnp.concatenate([pos2d.reshape(E), padpos.reshape(EPAD)])
  srcC, dstC, wC = _sc_scatter_meta(src1, dst1, w1, pos1)

  x = emb_table
  layers = [
      (rel_w0, root_w0, bias0, ln1_g, ln1_b, True),
      (rel_w1, root_w1, bias1, ln2_g, ln2_b, True),
      (rel_w2, root_w2, bias2, ln2_g, ln2_b, False),
  ]
  for relw, rootw, bias, g, b2, do_ln in layers:
    A = _sc_aggregate(x, srcC, dstC, wC, cntC)
    x = _tc_layer_call(A, x, relw, rootw.reshape(D, D),
                       bias.reshape(1, D), g.reshape(1, D),
                       b2.reshape(1, D), do_ln)

  idx = jnp.concatenate([user_indices, item_indices]).astype(jnp.int32)
  ui = _sc_gather_rows(x, idx)
  out = _tc_head_call(
      ui, mlp_w0, mlp_b0.reshape(1, -1), mlp_w1, mlp_b1.reshape(1, -1),
      mlp_w2, mlp_b2.reshape(1, -1), out_w.reshape(1, -1),
      out_b.reshape(1, 1))
  return out.reshape(B)


# speedup vs baseline: 1.8011x; 1.7881x over previous
import functools

import jax
import jax.numpy as jnp
from jax import lax
from jax.experimental import pallas as pl
from jax.experimental.pallas import tpu as pltpu
from jax.experimental.pallas import tpu_sc as plsc

N = 10000
D = 128
E = 320000
NC = 2
NS = 16
EPW = E // NS
CH = 80
NCHUNK = EPW // CH
NP = 10240
RPW = NP // NS
ZR = 128
B = 4096

_MESH = plsc.VectorSubcoreMesh(
    core_axis_name="c", subcore_axis_name="s", num_cores=NC, num_subcores=NS)


@functools.partial(
    pl.kernel,
    out_type=jax.ShapeDtypeStruct((NC, NP, D), jnp.float32),
    mesh=_MESH,
    scratch_types=(
        [pltpu.VMEM((CH,), jnp.int32) for _ in range(2)]
        + [pltpu.VMEM((CH,), jnp.int32) for _ in range(2)]
        + [pltpu.VMEM((CH,), jnp.float32) for _ in range(2)]
        + [pltpu.VMEM((CH, D), jnp.float32) for _ in range(2)]
        + [
            pltpu.VMEM((ZR, D), jnp.float32),
            pltpu.VMEM_SHARED((NP, D), jnp.float32),
        ]
        + [pltpu.SemaphoreType.DMA for _ in range(6)]
    ),
)
def _sc_aggregate(x_hbm, src_hbm, dst_hbm, w_hbm, out_hbm,
                  sia, sib, dia, dib, wva, wvb, ra, rb,
                  stage_v, acc_sh, ga, gb, ma, mb, sa, sb):
  c = lax.axis_index("c")
  s = lax.axis_index("s")

  def zrow(i, carry):
    for j in range(D // 16):
      stage_v[i, pl.ds(j * 16, 16)] = jnp.zeros((16,), jnp.float32)
    return carry
  lax.fori_loop(0, ZR, zrow, 0)
  r0 = s * RPW
  for z in range(RPW // ZR):
    pltpu.sync_copy(stage_v, acc_sh.at[pl.ds(r0 + z * ZR, ZR)])
  plsc.subcore_barrier()

  e0 = s * EPW
  sets = ((sia, dia, wva, ra, ga, ma, sa),
          (sib, dib, wvb, rb, gb, mb, sb))

  def issue_meta(k, st):
    si_r, di_r, wv_r, _, _, msem, _ = st
    off = e0 + k * CH
    pltpu.async_copy(src_hbm.at[pl.ds(off, CH)], si_r, msem)
    pltpu.async_copy(dst_hbm.at[pl.ds(off, CH)], di_r, msem)
    pltpu.async_copy(w_hbm.at[pl.ds(c * E + off, CH)], wv_r, msem)

  def wait_meta(st):
    si_r, di_r, wv_r, _, _, msem, _ = st
    pltpu.make_async_copy(src_hbm.at[pl.ds(0, CH)], si_r, msem).wait()
    pltpu.make_async_copy(dst_hbm.at[pl.ds(0, CH)], di_r, msem).wait()
    pltpu.make_async_copy(w_hbm.at[pl.ds(0, CH)], wv_r, msem).wait()

  def issue_gather(st):
    si_r, _, _, rows_r, gsem, _, _ = st
    pltpu.async_copy(x_hbm.at[si_r], rows_r, gsem)

  def wait_gather(st):
    si_r, _, _, rows_r, gsem, _, _ = st
    pltpu.make_async_copy(x_hbm.at[si_r], rows_r, gsem).wait()

  def issue_scatter(st):
    _, di_r, _, rows_r, _, _, ssem = st
    pltpu.async_copy(rows_r, acc_sh.at[di_r], ssem, add=True)

  def wait_scatter(st):
    _, di_r, _, rows_r, _, _, ssem = st
    pltpu.make_async_copy(rows_r, acc_sh.at[di_r], ssem).wait()

  def scale(st):
    _, _, wv_r, rows_r, _, _, _ = st
    for g in range(CH // 16):
      wg = wv_r[pl.ds(g * 16, 16)]
      for i in range(16):
        ws = wg[i]
        e = g * 16 + i
        for j in range(D // 16):
          sl = pl.ds(j * 16, 16)
          rows_r[e, sl] = rows_r[e, sl] * ws

  issue_meta(0, sets[0])
  wait_meta(sets[0])
  issue_gather(sets[0])
  issue_meta(1, sets[1])

  def pair(p, carry):
    for h in range(2):
      k = 2 * p + h
      sA = sets[h]
      sB = sets[1 - h]
      wait_gather(sA)
      scale(sA)
      issue_scatter(sA)
      @pl.when(k + 2 < NCHUNK)
      def _():
        wait_scatter(sA)
        issue_meta(k + 2, sA)
      @pl.when(k + 1 < NCHUNK)
      def _():
        wait_meta(sB)
        issue_gather(sB)
    return carry
  lax.fori_loop(0, NCHUNK // 2, pair, 0)
  wait_scatter(sets[0])
  wait_scatter(sets[1])

  plsc.subcore_barrier()
  for z in range(RPW // ZR):
    rr = r0 + z * ZR
    pltpu.sync_copy(acc_sh.at[pl.ds(rr, ZR)], out_hbm.at[c, pl.ds(rr, ZR)])


_GB = B * 2 // (NC * NS)


@functools.partial(
    pl.kernel,
    out_type=jax.ShapeDtypeStruct((2 * B, D), jnp.float32),
    mesh=_MESH,
    scratch_types=[
        pltpu.VMEM((128,), jnp.int32),
        pltpu.VMEM((128, D), jnp.float32),
        pltpu.SemaphoreType.DMA,
    ],
)
def _sc_gather_rows(x_hbm, idx_hbm, out_hbm, idx_v, rows_v, sem):
  c = lax.axis_index("c")
  s = lax.axis_index("s")
  base = (s * NC + c) * _GB
  for t in range(_GB // 128):
    off = base + t * 128
    pltpu.sync_copy(idx_hbm.at[pl.ds(off, 128)], idx_v)
    pltpu.async_copy(x_hbm.at[idx_v], rows_v, sem).wait()
    pltpu.sync_copy(rows_v, out_hbm.at[pl.ds(off, 128)])


def _wsel_call(w2d, t2d):
  def body(w_ref, t_ref, o_ref):
    w = w_ref[...]
    t = t_ref[...]
    o_ref[0] = jnp.where(t == 0, w, 0.0)
    o_ref[1] = jnp.where(t == 1, w, 0.0)
  return pl.pallas_call(
      body,
      out_shape=jax.ShapeDtypeStruct((2, E // 128, 128), jnp.float32),
  )(w2d, t2d)


RB = 1000


def _tc_layer_call(A, x, relw, rootw, bias, g, b2, do_ln):
  def body(a_ref, x_ref, rw_ref, rootw_ref, bias_ref, g_ref, b2_ref, o_ref):
    y = jnp.dot(a_ref[0], rw_ref[0], preferred_element_type=jnp.float32)
    y = y + jnp.dot(a_ref[1], rw_ref[1], preferred_element_type=jnp.float32)
    y = y + jnp.dot(x_ref[...], rootw_ref[...],
                    preferred_element_type=jnp.float32)
    y = y + bias_ref[...]
    if do_ln:
      y = jnp.maximum(y, 0.0)
      m = jnp.mean(y, axis=-1, keepdims=True)
      yc = y - m
      v = jnp.mean(yc * yc, axis=-1, keepdims=True)
      y = yc * lax.rsqrt(v + 1e-5) * g_ref[...] + b2_ref[...]
    o_ref[...] = y
  return pl.pallas_call(
      body,
      grid=(N // RB,),
      in_specs=[
          pl.BlockSpec((2, RB, D), lambda i: (0, i, 0)),
          pl.BlockSpec((RB, D), lambda i: (i, 0)),
          pl.BlockSpec((2, D, D), lambda i: (0, 0, 0)),
          pl.BlockSpec((D, D), lambda i: (0, 0)),
          pl.BlockSpec((1, D), lambda i: (0, 0)),
          pl.BlockSpec((1, D), lambda i: (0, 0)),
          pl.BlockSpec((1, D), lambda i: (0, 0)),
      ],
      out_specs=pl.BlockSpec((RB, D), lambda i: (i, 0)),
      out_shape=jax.ShapeDtypeStruct((N, D), jnp.float32),
  )(A, x, relw, rootw, bias, g, b2)


def _tc_head_call(ui, w0, b0, w1, b1, w2, b2, owt, ob):
  def body(ui_ref, w0_ref, b0_ref, w1_ref, b1_ref, w2_ref, b2_ref,
           ow_ref, ob_ref, o_ref):
    u = ui_ref[:B]
    it = ui_ref[B:]
    h = (jnp.dot(u, w0_ref[:D], preferred_element_type=jnp.float32)
         + jnp.dot(it, w0_ref[D:], preferred_element_type=jnp.float32)
         + b0_ref[...])
    h = jnp.maximum(h, 0.0)
    h = jnp.maximum(
        jnp.dot(h, w1_ref[...], preferred_element_type=jnp.float32)
        + b1_ref[...], 0.0)
    h = jnp.maximum(
        jnp.dot(h, w2_ref[...], preferred_element_type=jnp.float32)
        + b2_ref[...], 0.0)
    nu = jnp.maximum(jnp.sqrt(jnp.sum(u * u, axis=-1, keepdims=True)), 1e-12)
    ni = jnp.maximum(jnp.sqrt(jnp.sum(it * it, axis=-1, keepdims=True)),
                     1e-12)
    gmf = (u / nu) * (it / ni)
    logit = (jnp.sum(gmf * ow_ref[:, :D], axis=-1, keepdims=True)
             + jnp.sum(h * ow_ref[:, D:], axis=-1, keepdims=True)
             + ob_ref[...])
    o_ref[...] = jax.nn.sigmoid(logit)
  return pl.pallas_call(
      body,
      out_shape=jax.ShapeDtypeStruct((B, 1), jnp.float32),
  )(ui, w0, b0, w1, b1, w2, b2, owt, ob)


def kernel(user_indices, item_indices, edge_index, edge_type, edge_weight,
           emb_table, rel_w0, rel_w1, rel_w2, root_w0, root_w1, root_w2,
           bias0, bias1, bias2, ln1_g, ln1_b, ln2_g, ln2_b,
           mlp_w0, mlp_b0, mlp_w1, mlp_b1, mlp_w2, mlp_b2, out_w, out_b):
  src1 = edge_index[0].astype(jnp.int32)
  dst1 = edge_index[1].astype(jnp.int32)
  t2d = edge_type.astype(jnp.int32).reshape(E // 128, 128)
  w2d = edge_weight.reshape(E // 128, 128)
  wsel = _wsel_call(w2d, t2d).reshape(NC * E)

  x = emb_table
  layers = [
      (rel_w0, root_w0, bias0, ln1_g, ln1_b, True),
      (rel_w1, root_w1, bias1, ln2_g, ln2_b, True),
      (rel_w2, root_w2, bias2, ln2_g, ln2_b, False),
  ]
  for relw, rootw, bias, g, b2, do_ln in layers:
    A = _sc_aggregate(x, src1, dst1, wsel)
    x = _tc_layer_call(A, x, relw, rootw.reshape(D, D),
                       bias.reshape(1, D), g.reshape(1, D),
                       b2.reshape(1, D), do_ln)

  idx = jnp.concatenate([user_indices, item_indices]).astype(jnp.int32)
  ui = _sc_gather_rows(x, idx)
  out = _tc_head_call(
      ui, mlp_w0, mlp_b0.reshape(1, -1), mlp_w1, mlp_b1.reshape(1, -1),
      mlp_w2, mlp_b2.reshape(1, -1), out_w.reshape(1, -1),
      out_b.reshape(1, 1))
  return out.reshape(B)
